# Initial kernel scaffold; baseline (speedup 1.0000x reference)
import jax
import jax.numpy as jnp
from jax.experimental import pallas as pl


def kernel(x, edge_index, batch, Wl0, Wr0, att0, b0, Wl1, Wr1, att1, b1, Wl2, Wr2, att2, b2, Wl3, Wr3, att3, b3):
    raise NotImplementedError("write your pallas kernel here")


# speedup vs baseline: 12.7429x; 12.7429x over previous
import functools

import jax
import jax.numpy as jnp
from jax import lax
from jax.experimental import pallas as pl
from jax.experimental.pallas import tpu as pltpu
from jax.experimental.pallas import tpu_sc as plsc

N = 10000
E = 320000
G = 64
IN_FEAT = 128
F = 192
TW = 128
WROW = 144
DEN_COL = 128
TW3 = 32
WROW3 = 48
DEN_COL3 = 32
NC = 2
NS = 16
NW = NC * NS
CH = 80
RPT = N // NS
BN = 1000


def _edge_kernel(featsplit):
  if featsplit:
    ns, sv, tw, wrow, den_col, ept = 2, 4, TW, WROW, DEN_COL, E // NS
  else:
    ns, sv, tw, wrow, den_col, ept = 1, 2, TW3, WROW3, DEN_COL3, E // NW
  nf = ns * sv
  nchunk = ept // CH
  att_shape = (NC, tw) if featsplit else (tw,)
  mesh = plsc.VectorSubcoreMesh(core_axis_name="c", subcore_axis_name="s")

  scratch = [
      pltpu.VMEM((CH,), jnp.int32),
      pltpu.VMEM((CH,), jnp.int32),
      pltpu.VMEM((CH,), jnp.int32),
      pltpu.VMEM((CH,), jnp.int32),
      pltpu.VMEM((CH, tw), jnp.float32),
      pltpu.VMEM((CH, tw), jnp.float32),
      pltpu.VMEM((CH, wrow), jnp.float32),
      pltpu.VMEM((tw,), jnp.float32),
      pltpu.VMEM_SHARED((N, wrow), jnp.float32),
      pltpu.SemaphoreType.DMA,
      pltpu.SemaphoreType.DMA,
  ]

  @functools.partial(
      pl.kernel, mesh=mesh,
      compiler_params=pltpu.CompilerParams(
          use_tc_tiling_on_sc=False, needs_layout_passes=False),
      out_type=jax.ShapeDtypeStruct((NC, N, wrow), jnp.float32),
      scratch_types=scratch,
  )
  def body(tl_hbm, tr_hbm, src_hbm, dst_hbm, att_hbm, out_hbm,
           src_v, dst_v, sidx_v, didx_v, xlt, xrt, wbuf, att_v,
           acc_sh, sem1, sem2):
    cid = lax.axis_index("c")
    sid = lax.axis_index("s")

    if featsplit:
      pltpu.sync_copy(att_hbm.at[cid], att_v)
    else:
      pltpu.sync_copy(att_hbm, att_v)

    zeros16 = jnp.zeros((16,), jnp.float32)

    def zrow(r, carry):
      for j in range(wrow // 16):
        wbuf[r, pl.ds(16 * j, 16)] = zeros16
      return carry
    lax.fori_loop(0, CH, zrow, 0)
    for q in range(RPT // CH):
      pltpu.sync_copy(wbuf, acc_sh.at[pl.ds(sid * RPT + q * CH, CH)])
    pltpu.sync_copy(wbuf.at[pl.ds(0, RPT % CH)],
                    acc_sh.at[pl.ds(sid * RPT + RPT - RPT % CH, RPT % CH)])
    plsc.subcore_barrier()

    att_regs = [att_v[pl.ds(16 * j, 16)] for j in range(nf)]
    lane = lax.broadcasted_iota(jnp.int32, (16,), 0)

    def chunk_body(k, carry):
      if featsplit:
        base = sid * ept + k * CH
      else:
        base = (sid * NC + cid) * ept + k * CH
      pltpu.sync_copy(src_hbm.at[pl.ds(base, CH)], src_v)
      pltpu.sync_copy(dst_hbm.at[pl.ds(base, CH)], dst_v)
      if featsplit:
        off = cid * N
        for g in range(CH // 16):
          sl = pl.ds(16 * g, 16)
          sidx_v[sl] = src_v[sl] + off
          didx_v[sl] = dst_v[sl] + off
        pltpu.async_copy(tl_hbm.at[sidx_v], xlt, sem1).wait()
        pltpu.async_copy(tr_hbm.at[didx_v], xrt, sem2).wait()
      else:
        pltpu.async_copy(tl_hbm.at[src_v], xlt, sem1).wait()
        pltpu.async_copy(tr_hbm.at[dst_v], xrt, sem2).wait()

      def edge_body(e, c2):
        dvec = zeros16
        for s in range(ns):
          acc = zeros16
          xls = []
          for t in range(sv):
            j = s * sv + t
            a = xlt[e, pl.ds(16 * j, 16)]
            b = xrt[e, pl.ds(16 * j, 16)]
            u = a + b
            lr = jnp.maximum(u, 0.2 * u)
            acc = acc + lr * att_regs[j]
            xls.append(a)
          logit = jnp.sum(acc)
          exv = jnp.exp(jnp.full((16,), logit, jnp.float32))
          for t in range(sv):
            j = s * sv + t
            wbuf[e, pl.ds(16 * j, 16)] = xls[t] * exv
          if featsplit:
            den_lane = cid * ns + s
          else:
            den_lane = s
          dvec = dvec + jnp.where(lane == den_lane, exv, 0.0)
        wbuf[e, pl.ds(den_col, 16)] = dvec
        return c2

      lax.fori_loop(0, CH, edge_body, 0)
      pltpu.sync_copy(wbuf, acc_sh.at[dst_v], add=True)
      return carry

    lax.fori_loop(0, nchunk, chunk_body, 0)

    plsc.subcore_barrier()
    pltpu.sync_copy(acc_sh.at[pl.ds(sid * RPT, RPT)],
                    out_hbm.at[cid, pl.ds(sid * RPT, RPT)])

  return body


_edge3 = _edge_kernel(True)
_edge1 = _edge_kernel(False)


def _split_tables(y_ref, o_ref):
  y = y_ref
  o_ref[0] = y[:, :TW]
  o_ref[1] = jnp.concatenate(
      [y[:, TW:F], jnp.zeros((y.shape[0], 2 * TW - F), jnp.float32)], axis=1)


def _mm0_body(x_ref, wl_ref, wr_ref, ol_ref, or_ref):
  xb = x_ref[...]
  _split_tables(jnp.dot(xb, wl_ref[...], preferred_element_type=jnp.float32),
                ol_ref)
  _split_tables(jnp.dot(xb, wr_ref[...], preferred_element_type=jnp.float32),
                or_ref)


def _mm0(x, wl, wr):
  return pl.pallas_call(
      _mm0_body,
      grid=(N // BN,),
      in_specs=[
          pl.BlockSpec((BN, IN_FEAT), lambda i: (i, 0)),
          pl.BlockSpec((IN_FEAT, F), lambda i: (0, 0)),
          pl.BlockSpec((IN_FEAT, F), lambda i: (0, 0)),
      ],
      out_specs=[
          pl.BlockSpec((NC, BN, TW), lambda i: (0, i, 0)),
          pl.BlockSpec((NC, BN, TW), lambda i: (0, i, 0)),
      ],
      out_shape=[
          jax.ShapeDtypeStruct((NC, N, TW), jnp.float32),
          jax.ShapeDtypeStruct((NC, N, TW), jnp.float32),
      ],
  )(x, wl, wr)


def _normalize(acc_ref, b_ref):
  a0 = acc_ref[0]
  a1 = acc_ref[1]
  f = jnp.concatenate([a0[:, :TW], a1[:, :F - TW]], axis=1)
  d3 = a0[:, DEN_COL:DEN_COL + 3] + a1[:, DEN_COL:DEN_COL + 3]
  r3 = lax.broadcasted_iota(jnp.int32, (3, F), 0)
  c3 = lax.broadcasted_iota(jnp.int32, (3, F), 1) // 64
  sel = (r3 == c3).astype(jnp.float32)
  den = jnp.dot(d3, sel, preferred_element_type=jnp.float32)
  return f / (den + 1e-16) + b_ref[...]


def _comb_mm_body(acc_ref, b_ref, wl_ref, wr_ref, ol_ref, or_ref):
  xb = _normalize(acc_ref, b_ref)
  _split_tables(jnp.dot(xb, wl_ref[...], preferred_element_type=jnp.float32),
                ol_ref)
  _split_tables(jnp.dot(xb, wr_ref[...], preferred_element_type=jnp.float32),
                or_ref)


def _comb_mm(acc, b, wl, wr):
  return pl.pallas_call(
      _comb_mm_body,
      grid=(N // BN,),
      in_specs=[
          pl.BlockSpec((NC, BN, WROW), lambda i: (0, i, 0)),
          pl.BlockSpec((1, F), lambda i: (0, 0)),
          pl.BlockSpec((F, F), lambda i: (0, 0)),
          pl.BlockSpec((F, F), lambda i: (0, 0)),
      ],
      out_specs=[
          pl.BlockSpec((NC, BN, TW), lambda i: (0, i, 0)),
          pl.BlockSpec((NC, BN, TW), lambda i: (0, i, 0)),
      ],
      out_shape=[
          jax.ShapeDtypeStruct((NC, N, TW), jnp.float32),
          jax.ShapeDtypeStruct((NC, N, TW), jnp.float32),
      ],
  )(acc, b, wl, wr)


def _comb_mm3_body(acc_ref, b_ref, wl_ref, wr_ref, ol_ref, or_ref):
  xb = _normalize(acc_ref, b_ref)
  ol_ref[...] = jnp.dot(xb, wl_ref[...], preferred_element_type=jnp.float32)
  or_ref[...] = jnp.dot(xb, wr_ref[...], preferred_element_type=jnp.float32)


def _comb_mm3(acc, b, wl, wr):
  return pl.pallas_call(
      _comb_mm3_body,
      grid=(N // BN,),
      in_specs=[
          pl.BlockSpec((NC, BN, WROW), lambda i: (0, i, 0)),
          pl.BlockSpec((1, F), lambda i: (0, 0)),
          pl.BlockSpec((F, TW3), lambda i: (0, 0)),
          pl.BlockSpec((F, TW3), lambda i: (0, 0)),
      ],
      out_specs=[
          pl.BlockSpec((BN, TW3), lambda i: (i, 0)),
          pl.BlockSpec((BN, TW3), lambda i: (i, 0)),
      ],
      out_shape=[
          jax.ShapeDtypeStruct((N, TW3), jnp.float32),
          jax.ShapeDtypeStruct((N, TW3), jnp.float32),
      ],
  )(acc, b, wl, wr)


def _final_body(acc_ref, b_ref, batch_ref, out_ref):
  i = pl.program_id(0)
  a0 = acc_ref[0]
  a1 = acc_ref[1]
  f = a0[:, :20] + a1[:, :20]
  den = a0[:, DEN_COL3:DEN_COL3 + 1] + a1[:, DEN_COL3:DEN_COL3 + 1]
  h = f / (den + 1e-16) + b_ref[...]
  bt = batch_ref[0]
  oh = (lax.broadcasted_iota(jnp.int32, (G, BN), 0) == bt).astype(jnp.float32)
  p = jnp.dot(oh, h, preferred_element_type=jnp.float32)

  @pl.when(i == 0)
  def _():
    out_ref[...] = jnp.zeros_like(out_ref)

  out_ref[...] += p


def _final(acc, b, batch_r):
  return pl.pallas_call(
      _final_body,
      grid=(N // BN,),
      in_specs=[
          pl.BlockSpec((NC, BN, WROW3), lambda i: (0, i, 0)),
          pl.BlockSpec((1, 20), lambda i: (0, 0)),
          pl.BlockSpec((1, 1, BN), lambda i: (i, 0, 0)),
      ],
      out_specs=pl.BlockSpec((G, 20), lambda i: (0, 0)),
      out_shape=jax.ShapeDtypeStruct((G, 20), jnp.float32),
  )(acc, b, batch_r)


def _att_split(att):
  a = att.reshape(-1)
  return jnp.stack([a[:TW], jnp.pad(a[TW:], (0, 2 * TW - F))])


def kernel(x, edge_index, batch, Wl0, Wr0, att0, b0, Wl1, Wr1, att1, b1,
           Wl2, Wr2, att2, b2, Wl3, Wr3, att3, b3):
  src = edge_index[0]
  dst = edge_index[1]

  tl0, tr0 = _mm0(x, Wl0, Wr0)
  acc0 = _edge3(tl0.reshape(NC * N, TW), tr0.reshape(NC * N, TW),
                src, dst, _att_split(att0))

  tl1, tr1 = _comb_mm(acc0, b0.reshape(1, -1), Wl1, Wr1)
  acc1 = _edge3(tl1.reshape(NC * N, TW), tr1.reshape(NC * N, TW),
                src, dst, _att_split(att1))

  tl2, tr2 = _comb_mm(acc1, b1.reshape(1, -1), Wl2, Wr2)
  acc2 = _edge3(tl2.reshape(NC * N, TW), tr2.reshape(NC * N, TW),
                src, dst, _att_split(att2))

  wl3 = jnp.pad(Wl3, ((0, 0), (0, TW3 - 20)))
  wr3 = jnp.pad(Wr3, ((0, 0), (0, TW3 - 20)))
  xl3, xr3 = _comb_mm3(acc2, b2.reshape(1, -1), wl3, wr3)
  att3p = jnp.pad(att3.reshape(-1), (0, TW3 - 20))
  acc3 = _edge1(xl3, xr3, src, dst, att3p)

  return _final(acc3, b3.reshape(1, -1), batch.reshape(N // BN, 1, BN))


# speedup vs baseline: 14.0494x; 1.1025x over previous
import functools

import jax
import jax.numpy as jnp
from jax import lax
from jax.experimental import pallas as pl
from jax.experimental.pallas import tpu as pltpu
from jax.experimental.pallas import tpu_sc as plsc

N = 10000
E = 320000
G = 64
IN_FEAT = 128
F = 192
TW = 128
WROW = 144
DEN_COL = 128
TW3 = 32
WROW3 = 48
DEN_COL3 = 32
NC = 2
NS = 16
NW = NC * NS
CH = 80
RPT = N // NS
BN = 1000


def _edge_kernel(featsplit):
  if featsplit:
    ns, sv, tw, wrow, den_col, ept = 2, 4, TW, WROW, DEN_COL, E // NS
  else:
    ns, sv, tw, wrow, den_col, ept = 1, 2, TW3, WROW3, DEN_COL3, E // NW
  nf = ns * sv
  nchunk = ept // CH
  att_shape = (NC, tw) if featsplit else (tw,)
  mesh = plsc.VectorSubcoreMesh(core_axis_name="c", subcore_axis_name="s")

  scratch = [
      pltpu.VMEM((CH,), jnp.int32),
      pltpu.VMEM((CH,), jnp.int32),
      pltpu.VMEM((CH,), jnp.int32),
      pltpu.VMEM((CH,), jnp.int32),
      pltpu.VMEM((CH, tw), jnp.float32),
      pltpu.VMEM((CH, tw), jnp.float32),
      pltpu.VMEM((CH, wrow), jnp.float32),
      pltpu.VMEM((tw,), jnp.float32),
      pltpu.VMEM_SHARED((N, wrow), jnp.float32),
      pltpu.SemaphoreType.DMA,
      pltpu.SemaphoreType.DMA,
  ]

  @functools.partial(
      pl.kernel, mesh=mesh,
      compiler_params=pltpu.CompilerParams(
          use_tc_tiling_on_sc=False, needs_layout_passes=False),
      out_type=jax.ShapeDtypeStruct((NC, N, wrow), jnp.float32),
      scratch_types=scratch,
  )
  def body(tl_hbm, tr_hbm, src_hbm, dst_hbm, att_hbm, out_hbm,
           src_v, dst_v, sidx_v, didx_v, xlt, xrt, wbuf, att_v,
           acc_sh, sem1, sem2):
    cid = lax.axis_index("c")
    sid = lax.axis_index("s")

    if featsplit:
      pltpu.sync_copy(att_hbm.at[cid], att_v)
    else:
      pltpu.sync_copy(att_hbm, att_v)

    zeros16 = jnp.zeros((16,), jnp.float32)

    def zrow(r, carry):
      for j in range(wrow // 16):
        wbuf[r, pl.ds(16 * j, 16)] = zeros16
      return carry
    lax.fori_loop(0, CH, zrow, 0)
    for q in range(RPT // CH):
      pltpu.sync_copy(wbuf, acc_sh.at[pl.ds(sid * RPT + q * CH, CH)])
    pltpu.sync_copy(wbuf.at[pl.ds(0, RPT % CH)],
                    acc_sh.at[pl.ds(sid * RPT + RPT - RPT % CH, RPT % CH)])
    plsc.subcore_barrier()

    att_regs = [att_v[pl.ds(16 * j, 16)] for j in range(nf)]
    lane = lax.broadcasted_iota(jnp.int32, (16,), 0)

    def chunk_body(k, carry):
      if featsplit:
        base = sid * ept + k * CH
      else:
        base = (sid * NC + cid) * ept + k * CH
      pltpu.sync_copy(src_hbm.at[pl.ds(base, CH)], src_v)
      pltpu.sync_copy(dst_hbm.at[pl.ds(base, CH)], dst_v)
      if featsplit:
        off = cid * N
        for g in range(CH // 16):
          sl = pl.ds(16 * g, 16)
          sidx_v[sl] = src_v[sl] + off
          didx_v[sl] = dst_v[sl] + off
        c1 = pltpu.async_copy(tl_hbm.at[sidx_v], xlt, sem1)
        c2 = pltpu.async_copy(tr_hbm.at[didx_v], xrt, sem2)
        c1.wait()
        c2.wait()
      else:
        c1 = pltpu.async_copy(tl_hbm.at[src_v], xlt, sem1)
        c2 = pltpu.async_copy(tr_hbm.at[dst_v], xrt, sem2)
        c1.wait()
        c2.wait()

      def edge_body(e, c2):
        dvec = zeros16
        for s in range(ns):
          acc = zeros16
          xls = []
          for t in range(sv):
            j = s * sv + t
            a = xlt[e, pl.ds(16 * j, 16)]
            b = xrt[e, pl.ds(16 * j, 16)]
            u = a + b
            lr = jnp.maximum(u, 0.2 * u)
            acc = acc + lr * att_regs[j]
            xls.append(a)
          logit = jnp.sum(acc)
          exv = jnp.exp(jnp.full((16,), logit, jnp.float32))
          for t in range(sv):
            j = s * sv + t
            wbuf[e, pl.ds(16 * j, 16)] = xls[t] * exv
          if featsplit:
            den_lane = cid * ns + s
          else:
            den_lane = s
          dvec = dvec + jnp.where(lane == den_lane, exv, 0.0)
        wbuf[e, pl.ds(den_col, 16)] = dvec
        return c2

      lax.fori_loop(0, CH, edge_body, 0)
      pltpu.sync_copy(wbuf, acc_sh.at[dst_v], add=True)
      return carry

    lax.fori_loop(0, nchunk, chunk_body, 0)

    plsc.subcore_barrier()
    pltpu.sync_copy(acc_sh.at[pl.ds(sid * RPT, RPT)],
                    out_hbm.at[cid, pl.ds(sid * RPT, RPT)])

  return body


_edge3 = _edge_kernel(True)
_edge1 = _edge_kernel(False)


def _split_tables(y_ref, o_ref):
  y = y_ref
  o_ref[0] = y[:, :TW]
  o_ref[1] = jnp.concatenate(
      [y[:, TW:F], jnp.zeros((y.shape[0], 2 * TW - F), jnp.float32)], axis=1)


def _mm0_body(x_ref, wl_ref, wr_ref, ol_ref, or_ref):
  xb = x_ref[...]
  _split_tables(jnp.dot(xb, wl_ref[...], preferred_element_type=jnp.float32),
                ol_ref)
  _split_tables(jnp.dot(xb, wr_ref[...], preferred_element_type=jnp.float32),
                or_ref)


def _mm0(x, wl, wr):
  return pl.pallas_call(
      _mm0_body,
      grid=(N // BN,),
      in_specs=[
          pl.BlockSpec((BN, IN_FEAT), lambda i: (i, 0)),
          pl.BlockSpec((IN_FEAT, F), lambda i: (0, 0)),
          pl.BlockSpec((IN_FEAT, F), lambda i: (0, 0)),
      ],
      out_specs=[
          pl.BlockSpec((NC, BN, TW), lambda i: (0, i, 0)),
          pl.BlockSpec((NC, BN, TW), lambda i: (0, i, 0)),
      ],
      out_shape=[
          jax.ShapeDtypeStruct((NC, N, TW), jnp.float32),
          jax.ShapeDtypeStruct((NC, N, TW), jnp.float32),
      ],
  )(x, wl, wr)


def _normalize(acc_ref, b_ref):
  a0 = acc_ref[0]
  a1 = acc_ref[1]
  f = jnp.concatenate([a0[:, :TW], a1[:, :F - TW]], axis=1)
  d3 = a0[:, DEN_COL:DEN_COL + 3] + a1[:, DEN_COL:DEN_COL + 3]
  r3 = lax.broadcasted_iota(jnp.int32, (3, F), 0)
  c3 = lax.broadcasted_iota(jnp.int32, (3, F), 1) // 64
  sel = (r3 == c3).astype(jnp.float32)
  den = jnp.dot(d3, sel, preferred_element_type=jnp.float32)
  return f / (den + 1e-16) + b_ref[...]


def _comb_mm_body(acc_ref, b_ref, wl_ref, wr_ref, ol_ref, or_ref):
  xb = _normalize(acc_ref, b_ref)
  _split_tables(jnp.dot(xb, wl_ref[...], preferred_element_type=jnp.float32),
                ol_ref)
  _split_tables(jnp.dot(xb, wr_ref[...], preferred_element_type=jnp.float32),
                or_ref)


def _comb_mm(acc, b, wl, wr):
  return pl.pallas_call(
      _comb_mm_body,
      grid=(N // BN,),
      in_specs=[
          pl.BlockSpec((NC, BN, WROW), lambda i: (0, i, 0)),
          pl.BlockSpec((1, F), lambda i: (0, 0)),
          pl.BlockSpec((F, F), lambda i: (0, 0)),
          pl.BlockSpec((F, F), lambda i: (0, 0)),
      ],
      out_specs=[
          pl.BlockSpec((NC, BN, TW), lambda i: (0, i, 0)),
          pl.BlockSpec((NC, BN, TW), lambda i: (0, i, 0)),
      ],
      out_shape=[
          jax.ShapeDtypeStruct((NC, N, TW), jnp.float32),
          jax.ShapeDtypeStruct((NC, N, TW), jnp.float32),
      ],
  )(acc, b, wl, wr)


def _comb_mm3_body(acc_ref, b_ref, wl_ref, wr_ref, ol_ref, or_ref):
  xb = _normalize(acc_ref, b_ref)
  ol_ref[...] = jnp.dot(xb, wl_ref[...], preferred_element_type=jnp.float32)
  or_ref[...] = jnp.dot(xb, wr_ref[...], preferred_element_type=jnp.float32)


def _comb_mm3(acc, b, wl, wr):
  return pl.pallas_call(
      _comb_mm3_body,
      grid=(N // BN,),
      in_specs=[
          pl.BlockSpec((NC, BN, WROW), lambda i: (0, i, 0)),
          pl.BlockSpec((1, F), lambda i: (0, 0)),
          pl.BlockSpec((F, TW3), lambda i: (0, 0)),
          pl.BlockSpec((F, TW3), lambda i: (0, 0)),
      ],
      out_specs=[
          pl.BlockSpec((BN, TW3), lambda i: (i, 0)),
          pl.BlockSpec((BN, TW3), lambda i: (i, 0)),
      ],
      out_shape=[
          jax.ShapeDtypeStruct((N, TW3), jnp.float32),
          jax.ShapeDtypeStruct((N, TW3), jnp.float32),
      ],
  )(acc, b, wl, wr)


def _final_body(acc_ref, b_ref, batch_ref, out_ref):
  i = pl.program_id(0)
  a0 = acc_ref[0]
  a1 = acc_ref[1]
  f = a0[:, :20] + a1[:, :20]
  den = a0[:, DEN_COL3:DEN_COL3 + 1] + a1[:, DEN_COL3:DEN_COL3 + 1]
  h = f / (den + 1e-16) + b_ref[...]
  bt = batch_ref[0]
  oh = (lax.broadcasted_iota(jnp.int32, (G, BN), 0) == bt).astype(jnp.float32)
  p = jnp.dot(oh, h, preferred_element_type=jnp.float32)

  @pl.when(i == 0)
  def _():
    out_ref[...] = jnp.zeros_like(out_ref)

  out_ref[...] += p


def _final(acc, b, batch_r):
  return pl.pallas_call(
      _final_body,
      grid=(N // BN,),
      in_specs=[
          pl.BlockSpec((NC, BN, WROW3), lambda i: (0, i, 0)),
          pl.BlockSpec((1, 20), lambda i: (0, 0)),
          pl.BlockSpec((1, 1, BN), lambda i: (i, 0, 0)),
      ],
      out_specs=pl.BlockSpec((G, 20), lambda i: (0, 0)),
      out_shape=jax.ShapeDtypeStruct((G, 20), jnp.float32),
  )(acc, b, batch_r)


def _att_split(att):
  a = att.reshape(-1)
  return jnp.stack([a[:TW], jnp.pad(a[TW:], (0, 2 * TW - F))])


def kernel(x, edge_index, batch, Wl0, Wr0, att0, b0, Wl1, Wr1, att1, b1,
           Wl2, Wr2, att2, b2, Wl3, Wr3, att3, b3):
  src = edge_index[0]
  dst = edge_index[1]

  tl0, tr0 = _mm0(x, Wl0, Wr0)
  acc0 = _edge3(tl0.reshape(NC * N, TW), tr0.reshape(NC * N, TW),
                src, dst, _att_split(att0))

  tl1, tr1 = _comb_mm(acc0, b0.reshape(1, -1), Wl1, Wr1)
  acc1 = _edge3(tl1.reshape(NC * N, TW), tr1.reshape(NC * N, TW),
                src, dst, _att_split(att1))

  tl2, tr2 = _comb_mm(acc1, b1.reshape(1, -1), Wl2, Wr2)
  acc2 = _edge3(tl2.reshape(NC * N, TW), tr2.reshape(NC * N, TW),
                src, dst, _att_split(att2))

  wl3 = jnp.pad(Wl3, ((0, 0), (0, TW3 - 20)))
  wr3 = jnp.pad(Wr3, ((0, 0), (0, TW3 - 20)))
  xl3, xr3 = _comb_mm3(acc2, b2.reshape(1, -1), wl3, wr3)
  att3p = jnp.pad(att3.reshape(-1), (0, TW3 - 20))
  acc3 = _edge1(xl3, xr3, src, dst, att3p)

  return _final(acc3, b3.reshape(1, -1), batch.reshape(N // BN, 1, BN))


# speedup vs baseline: 14.8308x; 1.0556x over previous
import functools

import jax
import jax.numpy as jnp
from jax import lax
from jax.experimental import pallas as pl
from jax.experimental.pallas import tpu as pltpu
from jax.experimental.pallas import tpu_sc as plsc

N = 10000
E = 320000
G = 64
IN_FEAT = 128
F = 192
TW = 128
WROW = 144
DEN_COL = 128
TW3 = 32
WROW3 = 48
DEN_COL3 = 32
NC = 2
NS = 16
NW = NC * NS
CH = 80
RPT = N // NS
BN = 1000


def _edge_kernel(featsplit):
  if featsplit:
    ns, sv, tw, wrow, den_col, ept = 2, 4, TW, WROW, DEN_COL, E // NS
  else:
    ns, sv, tw, wrow, den_col, ept = 1, 2, TW3, WROW3, DEN_COL3, E // NW
  nf = ns * sv
  nchunk = ept // CH
  att_shape = (NC, tw) if featsplit else (tw,)
  mesh = plsc.VectorSubcoreMesh(core_axis_name="c", subcore_axis_name="s")

  scratch = [
      pltpu.VMEM((CH,), jnp.int32),
      pltpu.VMEM((CH,), jnp.int32),
      pltpu.VMEM((CH,), jnp.int32),
      pltpu.VMEM((CH,), jnp.int32),
      pltpu.VMEM((CH, tw), jnp.float32),
      pltpu.VMEM((CH, tw), jnp.float32),
      pltpu.VMEM((CH, wrow), jnp.float32),
      pltpu.VMEM((tw,), jnp.float32),
      pltpu.VMEM_SHARED((N, wrow), jnp.float32),
      pltpu.SemaphoreType.DMA,
      pltpu.SemaphoreType.DMA,
  ]

  @functools.partial(
      pl.kernel, mesh=mesh,
      compiler_params=pltpu.CompilerParams(
          use_tc_tiling_on_sc=False, needs_layout_passes=False),
      out_type=jax.ShapeDtypeStruct((NC, N, wrow), jnp.float32),
      scratch_types=scratch,
  )
  def body(tl_hbm, tr_hbm, src_hbm, dst_hbm, att_hbm, out_hbm,
           src_v, dst_v, sidx_v, didx_v, xlt, xrt, wbuf, att_v,
           acc_sh, sem1, sem2):
    cid = lax.axis_index("c")
    sid = lax.axis_index("s")

    if featsplit:
      pltpu.sync_copy(att_hbm.at[cid], att_v)
    else:
      pltpu.sync_copy(att_hbm, att_v)

    zeros16 = jnp.zeros((16,), jnp.float32)

    def zrow(r, carry):
      for j in range(wrow // 16):
        wbuf[r, pl.ds(16 * j, 16)] = zeros16
      return carry
    lax.fori_loop(0, CH, zrow, 0)
    for q in range(RPT // CH):
      pltpu.sync_copy(wbuf, acc_sh.at[pl.ds(sid * RPT + q * CH, CH)])
    pltpu.sync_copy(wbuf.at[pl.ds(0, RPT % CH)],
                    acc_sh.at[pl.ds(sid * RPT + RPT - RPT % CH, RPT % CH)])
    plsc.subcore_barrier()

    att_regs = [att_v[pl.ds(16 * j, 16)] for j in range(nf)]
    lane = lax.broadcasted_iota(jnp.int32, (16,), 0)

    def chunk_body(k, carry):
      if featsplit:
        base = sid * ept + k * CH
      else:
        base = (sid * NC + cid) * ept + k * CH
      i1 = pltpu.async_copy(src_hbm.at[pl.ds(base, CH)], src_v, sem1)
      i2 = pltpu.async_copy(dst_hbm.at[pl.ds(base, CH)], dst_v, sem2)
      i1.wait()
      i2.wait()
      if featsplit:
        off = cid * N
        for g in range(CH // 16):
          sl = pl.ds(16 * g, 16)
          sidx_v[sl] = src_v[sl] + off
          didx_v[sl] = dst_v[sl] + off
        c1 = pltpu.async_copy(tl_hbm.at[sidx_v], xlt, sem1)
        c2 = pltpu.async_copy(tr_hbm.at[didx_v], xrt, sem2)
        c1.wait()
        c2.wait()
      else:
        c1 = pltpu.async_copy(tl_hbm.at[src_v], xlt, sem1)
        c2 = pltpu.async_copy(tr_hbm.at[dst_v], xrt, sem2)
        c1.wait()
        c2.wait()

      def edge_body(e, c2):
        dvec = zeros16
        for s in range(ns):
          acc = zeros16
          xls = []
          for t in range(sv):
            j = s * sv + t
            a = xlt[e, pl.ds(16 * j, 16)]
            b = xrt[e, pl.ds(16 * j, 16)]
            u = a + b
            lr = jnp.maximum(u, 0.2 * u)
            acc = acc + lr * att_regs[j]
            xls.append(a)
          logit = jnp.sum(acc)
          exv = jnp.exp(jnp.full((16,), logit, jnp.float32))
          for t in range(sv):
            j = s * sv + t
            wbuf[e, pl.ds(16 * j, 16)] = xls[t] * exv
          if featsplit:
            den_lane = cid * ns + s
          else:
            den_lane = s
          dvec = dvec + jnp.where(lane == den_lane, exv, 0.0)
        wbuf[e, pl.ds(den_col, 16)] = dvec
        return c2

      lax.fori_loop(0, CH, edge_body, 0)
      pltpu.sync_copy(wbuf, acc_sh.at[dst_v], add=True)
      return carry

    lax.fori_loop(0, nchunk, chunk_body, 0)

    plsc.subcore_barrier()
    pltpu.sync_copy(acc_sh.at[pl.ds(sid * RPT, RPT)],
                    out_hbm.at[cid, pl.ds(sid * RPT, RPT)])

  return body


_edge3 = _edge_kernel(True)
_edge1 = _edge_kernel(False)


def _split_tables(y_ref, o_ref):
  y = y_ref
  o_ref[0] = y[:, :TW]
  o_ref[1] = jnp.concatenate(
      [y[:, TW:F], jnp.zeros((y.shape[0], 2 * TW - F), jnp.float32)], axis=1)


def _mm0_body(x_ref, wl_ref, wr_ref, ol_ref, or_ref):
  xb = x_ref[...]
  _split_tables(jnp.dot(xb, wl_ref[...], preferred_element_type=jnp.float32),
                ol_ref)
  _split_tables(jnp.dot(xb, wr_ref[...], preferred_element_type=jnp.float32),
                or_ref)


def _mm0(x, wl, wr):
  return pl.pallas_call(
      _mm0_body,
      grid=(N // BN,),
      in_specs=[
          pl.BlockSpec((BN, IN_FEAT), lambda i: (i, 0)),
          pl.BlockSpec((IN_FEAT, F), lambda i: (0, 0)),
          pl.BlockSpec((IN_FEAT, F), lambda i: (0, 0)),
      ],
      out_specs=[
          pl.BlockSpec((NC, BN, TW), lambda i: (0, i, 0)),
          pl.BlockSpec((NC, BN, TW), lambda i: (0, i, 0)),
      ],
      out_shape=[
          jax.ShapeDtypeStruct((NC, N, TW), jnp.float32),
          jax.ShapeDtypeStruct((NC, N, TW), jnp.float32),
      ],
  )(x, wl, wr)


def _normalize(acc_ref, b_ref):
  a0 = acc_ref[0]
  a1 = acc_ref[1]
  f = jnp.concatenate([a0[:, :TW], a1[:, :F - TW]], axis=1)
  d3 = a0[:, DEN_COL:DEN_COL + 3] + a1[:, DEN_COL:DEN_COL + 3]
  r3 = lax.broadcasted_iota(jnp.int32, (3, F), 0)
  c3 = lax.broadcasted_iota(jnp.int32, (3, F), 1) // 64
  sel = (r3 == c3).astype(jnp.float32)
  den = jnp.dot(d3, sel, preferred_element_type=jnp.float32)
  return f / (den + 1e-16) + b_ref[...]


def _comb_mm_body(acc_ref, b_ref, wl_ref, wr_ref, ol_ref, or_ref):
  xb = _normalize(acc_ref, b_ref)
  _split_tables(jnp.dot(xb, wl_ref[...], preferred_element_type=jnp.float32),
                ol_ref)
  _split_tables(jnp.dot(xb, wr_ref[...], preferred_element_type=jnp.float32),
                or_ref)


def _comb_mm(acc, b, wl, wr):
  return pl.pallas_call(
      _comb_mm_body,
      grid=(N // BN,),
      in_specs=[
          pl.BlockSpec((NC, BN, WROW), lambda i: (0, i, 0)),
          pl.BlockSpec((1, F), lambda i: (0, 0)),
          pl.BlockSpec((F, F), lambda i: (0, 0)),
          pl.BlockSpec((F, F), lambda i: (0, 0)),
      ],
      out_specs=[
          pl.BlockSpec((NC, BN, TW), lambda i: (0, i, 0)),
          pl.BlockSpec((NC, BN, TW), lambda i: (0, i, 0)),
      ],
      out_shape=[
          jax.ShapeDtypeStruct((NC, N, TW), jnp.float32),
          jax.ShapeDtypeStruct((NC, N, TW), jnp.float32),
      ],
  )(acc, b, wl, wr)


def _comb_mm3_body(acc_ref, b_ref, wl_ref, wr_ref, ol_ref, or_ref):
  xb = _normalize(acc_ref, b_ref)
  ol_ref[...] = jnp.dot(xb, wl_ref[...], preferred_element_type=jnp.float32)
  or_ref[...] = jnp.dot(xb, wr_ref[...], preferred_element_type=jnp.float32)


def _comb_mm3(acc, b, wl, wr):
  return pl.pallas_call(
      _comb_mm3_body,
      grid=(N // BN,),
      in_specs=[
          pl.BlockSpec((NC, BN, WROW), lambda i: (0, i, 0)),
          pl.BlockSpec((1, F), lambda i: (0, 0)),
          pl.BlockSpec((F, TW3), lambda i: (0, 0)),
          pl.BlockSpec((F, TW3), lambda i: (0, 0)),
      ],
      out_specs=[
          pl.BlockSpec((BN, TW3), lambda i: (i, 0)),
          pl.BlockSpec((BN, TW3), lambda i: (i, 0)),
      ],
      out_shape=[
          jax.ShapeDtypeStruct((N, TW3), jnp.float32),
          jax.ShapeDtypeStruct((N, TW3), jnp.float32),
      ],
  )(acc, b, wl, wr)


def _final_body(acc_ref, b_ref, batch_ref, out_ref):
  i = pl.program_id(0)
  a0 = acc_ref[0]
  a1 = acc_ref[1]
  f = a0[:, :20] + a1[:, :20]
  den = a0[:, DEN_COL3:DEN_COL3 + 1] + a1[:, DEN_COL3:DEN_COL3 + 1]
  h = f / (den + 1e-16) + b_ref[...]
  bt = batch_ref[0]
  oh = (lax.broadcasted_iota(jnp.int32, (G, BN), 0) == bt).astype(jnp.float32)
  p = jnp.dot(oh, h, preferred_element_type=jnp.float32)

  @pl.when(i == 0)
  def _():
    out_ref[...] = jnp.zeros_like(out_ref)

  out_ref[...] += p


def _final(acc, b, batch_r):
  return pl.pallas_call(
      _final_body,
      grid=(N // BN,),
      in_specs=[
          pl.BlockSpec((NC, BN, WROW3), lambda i: (0, i, 0)),
          pl.BlockSpec((1, 20), lambda i: (0, 0)),
          pl.BlockSpec((1, 1, BN), lambda i: (i, 0, 0)),
      ],
      out_specs=pl.BlockSpec((G, 20), lambda i: (0, 0)),
      out_shape=jax.ShapeDtypeStruct((G, 20), jnp.float32),
  )(acc, b, batch_r)


def _att_split(att):
  a = att.reshape(-1)
  return jnp.stack([a[:TW], jnp.pad(a[TW:], (0, 2 * TW - F))])


def kernel(x, edge_index, batch, Wl0, Wr0, att0, b0, Wl1, Wr1, att1, b1,
           Wl2, Wr2, att2, b2, Wl3, Wr3, att3, b3):
  src = edge_index[0]
  dst = edge_index[1]

  tl0, tr0 = _mm0(x, Wl0, Wr0)
  acc0 = _edge3(tl0.reshape(NC * N, TW), tr0.reshape(NC * N, TW),
                src, dst, _att_split(att0))

  tl1, tr1 = _comb_mm(acc0, b0.reshape(1, -1), Wl1, Wr1)
  acc1 = _edge3(tl1.reshape(NC * N, TW), tr1.reshape(NC * N, TW),
                src, dst, _att_split(att1))

  tl2, tr2 = _comb_mm(acc1, b1.reshape(1, -1), Wl2, Wr2)
  acc2 = _edge3(tl2.reshape(NC * N, TW), tr2.reshape(NC * N, TW),
                src, dst, _att_split(att2))

  wl3 = jnp.pad(Wl3, ((0, 0), (0, TW3 - 20)))
  wr3 = jnp.pad(Wr3, ((0, 0), (0, TW3 - 20)))
  xl3, xr3 = _comb_mm3(acc2, b2.reshape(1, -1), wl3, wr3)
  att3p = jnp.pad(att3.reshape(-1), (0, TW3 - 20))
  acc3 = _edge1(xl3, xr3, src, dst, att3p)

  return _final(acc3, b3.reshape(1, -1), batch.reshape(N // BN, 1, BN))


# speedup vs baseline: 14.8313x; 1.0000x over previous
import functools

import jax
import jax.numpy as jnp
from jax import lax
from jax.experimental import pallas as pl
from jax.experimental.pallas import tpu as pltpu
from jax.experimental.pallas import tpu_sc as plsc

N = 10000
E = 320000
G = 64
IN_FEAT = 128
F = 192
TW = 128
WROW = 144
DEN_COL = 128
TW3 = 32
WROW3 = 48
DEN_COL3 = 32
NC = 2
NS = 16
NW = NC * NS
CH = 80
RPT = N // NS
BN = 1000


def _edge_kernel(featsplit):
  if featsplit:
    ns, sv, tw, wrow, den_col, ept = 2, 4, TW, WROW, DEN_COL, E // NS
  else:
    ns, sv, tw, wrow, den_col, ept = 1, 2, TW3, WROW3, DEN_COL3, E // NW
  nf = ns * sv
  nchunk = ept // CH
  att_shape = (NC, tw) if featsplit else (tw,)
  mesh = plsc.VectorSubcoreMesh(core_axis_name="c", subcore_axis_name="s")

  scratch = [
      pltpu.VMEM((CH,), jnp.int32),
      pltpu.VMEM((CH,), jnp.int32),
      pltpu.VMEM((CH,), jnp.int32),
      pltpu.VMEM((CH,), jnp.int32),
      pltpu.VMEM((CH, tw), jnp.float32),
      pltpu.VMEM((CH, tw), jnp.float32),
      pltpu.VMEM((CH, wrow), jnp.float32),
      pltpu.VMEM((tw,), jnp.float32),
      pltpu.VMEM_SHARED((N, wrow), jnp.float32),
      pltpu.SemaphoreType.DMA,
      pltpu.SemaphoreType.DMA,
  ]

  @functools.partial(
      pl.kernel, mesh=mesh,
      compiler_params=pltpu.CompilerParams(
          use_tc_tiling_on_sc=False, needs_layout_passes=False),
      out_type=jax.ShapeDtypeStruct((NC, N, wrow), jnp.float32),
      scratch_types=scratch,
  )
  def body(tl_hbm, tr_hbm, src_hbm, dst_hbm, att_hbm, out_hbm,
           src_v, dst_v, sidx_v, didx_v, xlt, xrt, wbuf, att_v,
           acc_sh, sem1, sem2):
    cid = lax.axis_index("c")
    sid = lax.axis_index("s")

    if featsplit:
      pltpu.sync_copy(att_hbm.at[cid], att_v)
    else:
      pltpu.sync_copy(att_hbm, att_v)

    zeros16 = jnp.zeros((16,), jnp.float32)

    def zrow(r, carry):
      for j in range(wrow // 16):
        wbuf[r, pl.ds(16 * j, 16)] = zeros16
      return carry
    lax.fori_loop(0, CH, zrow, 0)
    for q in range(RPT // CH):
      pltpu.sync_copy(wbuf, acc_sh.at[pl.ds(sid * RPT + q * CH, CH)])
    pltpu.sync_copy(wbuf.at[pl.ds(0, RPT % CH)],
                    acc_sh.at[pl.ds(sid * RPT + RPT - RPT % CH, RPT % CH)])
    plsc.subcore_barrier()

    att_regs = [att_v[pl.ds(16 * j, 16)] for j in range(nf)]
    lane = lax.broadcasted_iota(jnp.int32, (16,), 0)

    def chunk_body(k, carry):
      if featsplit:
        base = sid * ept + k * CH
      else:
        base = (sid * NC + cid) * ept + k * CH
      i1 = pltpu.async_copy(src_hbm.at[pl.ds(base, CH)], src_v, sem1)
      i2 = pltpu.async_copy(dst_hbm.at[pl.ds(base, CH)], dst_v, sem2)
      i1.wait()
      i2.wait()
      if featsplit:
        off = cid * N
        for g in range(CH // 16):
          sl = pl.ds(16 * g, 16)
          sidx_v[sl] = src_v[sl] + off
          didx_v[sl] = dst_v[sl] + off
        c1 = pltpu.async_copy(tl_hbm.at[sidx_v], xlt, sem1)
        c2 = pltpu.async_copy(tr_hbm.at[didx_v], xrt, sem2)
        c1.wait()
        c2.wait()
      else:
        c1 = pltpu.async_copy(tl_hbm.at[src_v], xlt, sem1)
        c2 = pltpu.async_copy(tr_hbm.at[dst_v], xrt, sem2)
        c1.wait()
        c2.wait()

      def edge_body(e, c2):
        dvec = zeros16
        for s in range(ns):
          acc = zeros16
          xls = []
          for t in range(sv):
            j = s * sv + t
            a = xlt[e, pl.ds(16 * j, 16)]
            b = xrt[e, pl.ds(16 * j, 16)]
            u = a + b
            lr = jnp.maximum(u, 0.2 * u)
            acc = acc + lr * att_regs[j]
            xls.append(a)
          logit = jnp.sum(acc)
          exv = jnp.exp(jnp.full((16,), logit, jnp.float32))
          for t in range(sv):
            j = s * sv + t
            wbuf[e, pl.ds(16 * j, 16)] = xls[t] * exv
          if featsplit:
            den_lane = cid * ns + s
          else:
            den_lane = s
          dvec = dvec + jnp.where(lane == den_lane, exv, 0.0)
        wbuf[e, pl.ds(den_col, 16)] = dvec
        return c2

      lax.fori_loop(0, CH, edge_body, 0)
      pltpu.sync_copy(wbuf, acc_sh.at[dst_v], add=True)
      return carry

    lax.fori_loop(0, nchunk, chunk_body, 0)

    plsc.subcore_barrier()
    pltpu.sync_copy(acc_sh.at[pl.ds(sid * RPT, RPT)],
                    out_hbm.at[cid, pl.ds(sid * RPT, RPT)])

  return body


_edge_cache = {}


def _edge(featsplit):
  if featsplit not in _edge_cache:
    _edge_cache[featsplit] = _edge_kernel(featsplit)
  return _edge_cache[featsplit]


def _split_tables(y_ref, o_ref):
  y = y_ref
  o_ref[0] = y[:, :TW]
  o_ref[1] = jnp.concatenate(
      [y[:, TW:F], jnp.zeros((y.shape[0], 2 * TW - F), jnp.float32)], axis=1)


def _mm0_body(x_ref, wl_ref, wr_ref, ol_ref, or_ref):
  xb = x_ref[...]
  _split_tables(jnp.dot(xb, wl_ref[...], preferred_element_type=jnp.float32),
                ol_ref)
  _split_tables(jnp.dot(xb, wr_ref[...], preferred_element_type=jnp.float32),
                or_ref)


def _mm0(x, wl, wr):
  return pl.pallas_call(
      _mm0_body,
      grid=(N // BN,),
      in_specs=[
          pl.BlockSpec((BN, IN_FEAT), lambda i: (i, 0)),
          pl.BlockSpec((IN_FEAT, F), lambda i: (0, 0)),
          pl.BlockSpec((IN_FEAT, F), lambda i: (0, 0)),
      ],
      out_specs=[
          pl.BlockSpec((NC, BN, TW), lambda i: (0, i, 0)),
          pl.BlockSpec((NC, BN, TW), lambda i: (0, i, 0)),
      ],
      out_shape=[
          jax.ShapeDtypeStruct((NC, N, TW), jnp.float32),
          jax.ShapeDtypeStruct((NC, N, TW), jnp.float32),
      ],
  )(x, wl, wr)


def _normalize(acc_ref, b_ref):
  a0 = acc_ref[0]
  a1 = acc_ref[1]
  f = jnp.concatenate([a0[:, :TW], a1[:, :F - TW]], axis=1)
  d3 = a0[:, DEN_COL:DEN_COL + 3] + a1[:, DEN_COL:DEN_COL + 3]
  r3 = lax.broadcasted_iota(jnp.int32, (3, F), 0)
  c3 = lax.broadcasted_iota(jnp.int32, (3, F), 1) // 64
  sel = (r3 == c3).astype(jnp.float32)
  den = jnp.dot(d3, sel, preferred_element_type=jnp.float32)
  return f / (den + 1e-16) + b_ref[...]


def _comb_mm_body(acc_ref, b_ref, wl_ref, wr_ref, ol_ref, or_ref):
  xb = _normalize(acc_ref, b_ref)
  _split_tables(jnp.dot(xb, wl_ref[...], preferred_element_type=jnp.float32),
                ol_ref)
  _split_tables(jnp.dot(xb, wr_ref[...], preferred_element_type=jnp.float32),
                or_ref)


def _comb_mm(acc, b, wl, wr):
  return pl.pallas_call(
      _comb_mm_body,
      grid=(N // BN,),
      in_specs=[
          pl.BlockSpec((NC, BN, WROW), lambda i: (0, i, 0)),
          pl.BlockSpec((1, F), lambda i: (0, 0)),
          pl.BlockSpec((F, F), lambda i: (0, 0)),
          pl.BlockSpec((F, F), lambda i: (0, 0)),
      ],
      out_specs=[
          pl.BlockSpec((NC, BN, TW), lambda i: (0, i, 0)),
          pl.BlockSpec((NC, BN, TW), lambda i: (0, i, 0)),
      ],
      out_shape=[
          jax.ShapeDtypeStruct((NC, N, TW), jnp.float32),
          jax.ShapeDtypeStruct((NC, N, TW), jnp.float32),
      ],
  )(acc, b, wl, wr)


def _comb_mm3_body(acc_ref, b_ref, wl_ref, wr_ref, ol_ref, or_ref):
  xb = _normalize(acc_ref, b_ref)
  ol_ref[...] = jnp.dot(xb, wl_ref[...], preferred_element_type=jnp.float32)
  or_ref[...] = jnp.dot(xb, wr_ref[...], preferred_element_type=jnp.float32)


def _comb_mm3(acc, b, wl, wr):
  return pl.pallas_call(
      _comb_mm3_body,
      grid=(N // BN,),
      in_specs=[
          pl.BlockSpec((NC, BN, WROW), lambda i: (0, i, 0)),
          pl.BlockSpec((1, F), lambda i: (0, 0)),
          pl.BlockSpec((F, TW3), lambda i: (0, 0)),
          pl.BlockSpec((F, TW3), lambda i: (0, 0)),
      ],
      out_specs=[
          pl.BlockSpec((BN, TW3), lambda i: (i, 0)),
          pl.BlockSpec((BN, TW3), lambda i: (i, 0)),
      ],
      out_shape=[
          jax.ShapeDtypeStruct((N, TW3), jnp.float32),
          jax.ShapeDtypeStruct((N, TW3), jnp.float32),
      ],
  )(acc, b, wl, wr)


def _final_body(acc_ref, b_ref, batch_ref, out_ref):
  i = pl.program_id(0)
  a0 = acc_ref[0]
  a1 = acc_ref[1]
  f = a0[:, :20] + a1[:, :20]
  den = a0[:, DEN_COL3:DEN_COL3 + 1] + a1[:, DEN_COL3:DEN_COL3 + 1]
  h = f / (den + 1e-16) + b_ref[...]
  bt = batch_ref[0]
  oh = (lax.broadcasted_iota(jnp.int32, (G, BN), 0) == bt).astype(jnp.float32)
  p = jnp.dot(oh, h, preferred_element_type=jnp.float32)

  @pl.when(i == 0)
  def _():
    out_ref[...] = jnp.zeros_like(out_ref)

  out_ref[...] += p


def _final(acc, b, batch_r):
  return pl.pallas_call(
      _final_body,
      grid=(N // BN,),
      in_specs=[
          pl.BlockSpec((NC, BN, WROW3), lambda i: (0, i, 0)),
          pl.BlockSpec((1, 20), lambda i: (0, 0)),
          pl.BlockSpec((1, 1, BN), lambda i: (i, 0, 0)),
      ],
      out_specs=pl.BlockSpec((G, 20), lambda i: (0, 0)),
      out_shape=jax.ShapeDtypeStruct((G, 20), jnp.float32),
  )(acc, b, batch_r)


def _att_split(att):
  a = att.reshape(-1)
  return jnp.stack([a[:TW], jnp.pad(a[TW:], (0, 2 * TW - F))])


def kernel(x, edge_index, batch, Wl0, Wr0, att0, b0, Wl1, Wr1, att1, b1,
           Wl2, Wr2, att2, b2, Wl3, Wr3, att3, b3):
  src = edge_index[0]
  dst = edge_index[1]

  tl0, tr0 = _mm0(x, Wl0, Wr0)
  acc0 = _edge(True)(tl0.reshape(NC * N, TW), tr0.reshape(NC * N, TW),
                src, dst, _att_split(att0))

  tl1, tr1 = _comb_mm(acc0, b0.reshape(1, -1), Wl1, Wr1)
  acc1 = _edge(True)(tl1.reshape(NC * N, TW), tr1.reshape(NC * N, TW),
                src, dst, _att_split(att1))

  tl2, tr2 = _comb_mm(acc1, b1.reshape(1, -1), Wl2, Wr2)
  acc2 = _edge(True)(tl2.reshape(NC * N, TW), tr2.reshape(NC * N, TW),
                src, dst, _att_split(att2))

  wl3 = jnp.pad(Wl3, ((0, 0), (0, TW3 - 20)))
  wr3 = jnp.pad(Wr3, ((0, 0), (0, TW3 - 20)))
  xl3, xr3 = _comb_mm3(acc2, b2.reshape(1, -1), wl3, wr3)
  att3p = jnp.pad(att3.reshape(-1), (0, TW3 - 20))
  acc3 = _edge(False)(xl3, xr3, src, dst, att3p)

  return _final(acc3, b3.reshape(1, -1), batch.reshape(N // BN, 1, BN))
